# baseline (speedup 1.0000x reference)
import functools

import jax
import jax.numpy as jnp
from jax import lax
from jax.experimental import pallas as pl
from jax.experimental.pallas import tpu as pltpu
from jax.experimental.pallas import tpu_sc as plsc


def _sigmoid(x):
    return 1.0 / (1.0 + jnp.exp(-x))


def _k1_body(x_ref, gate_ref, sg_ref, su_ref, sd_ref, logits_ref, hsx_ref, shared_ref):
    x = x_ref[...]
    dn = (((1,), (1,)), ((), ()))
    logits = lax.dot_general(x, gate_ref[...], dn, preferred_element_type=jnp.float32)
    logits_ref[...] = logits
    score = _sigmoid(jnp.max(logits, axis=1, keepdims=True))
    hsx_ref[...] = (x * score).astype(jnp.bfloat16)
    g = lax.dot_general(x, sg_ref[...], dn, preferred_element_type=jnp.float32)
    u = lax.dot_general(x, su_ref[...], dn, preferred_element_type=jnp.float32)
    h = g * _sigmoid(g) * u
    shared_ref[...] = lax.dot_general(
        h, sd_ref[...], dn, preferred_element_type=jnp.float32
    ).astype(jnp.bfloat16)


def _make_k3_body(TM):
    def _k3_body(step_t, step_e, step_lo, step_hi, x_ref, sh_ref, rg_ref, ru_ref,
                 rd_ref, out_ref):
        s = pl.program_id(0)
        lo = step_lo[s]
        hi = step_hi[s]
        dn = (((1,), (1,)), ((), ()))

        @pl.when(lo < hi)
        def _():
            x = x_ref[...].astype(jnp.float32)
            g = lax.dot_general(x, rg_ref[0], dn, preferred_element_type=jnp.float32)
            u = lax.dot_general(x, ru_ref[0], dn, preferred_element_type=jnp.float32)
            h = g * _sigmoid(g) * u
            y = (
                lax.dot_general(h, rd_ref[0], dn, preferred_element_type=jnp.float32)
                + sh_ref[...].astype(jnp.float32)
            )
            rows = lax.broadcasted_iota(jnp.int32, (x.shape[0], 1), 0)
            mask = (rows >= lo) & (rows < hi)
            out_ref[...] = jnp.where(mask, y, out_ref[...])

    return _k3_body


def kernel(hidden_states, gate_w, sg_w, su_w, sd_w, rg_w, ru_w, rd_w):
    B_, S_, H_ = hidden_states.shape
    E_ = gate_w.shape[0]
    Ish = sg_w.shape[0]
    Ir = rg_w.shape[1]
    T_ = B_ * S_

    TM1 = 256
    TM = 128
    NT = T_ // TM
    NSTEP = NT + E_ - 1

    hs2 = hidden_states.reshape(T_, H_)

    logits, hsx, shared = pl.pallas_call(
        _k1_body,
        grid=(T_ // TM1,),
        in_specs=[
            pl.BlockSpec((TM1, H_), lambda i: (i, 0)),
            pl.BlockSpec((E_, H_), lambda i: (0, 0)),
            pl.BlockSpec((Ish, H_), lambda i: (0, 0)),
            pl.BlockSpec((Ish, H_), lambda i: (0, 0)),
            pl.BlockSpec((H_, Ish), lambda i: (0, 0)),
        ],
        out_specs=[
            pl.BlockSpec((TM1, E_), lambda i: (i, 0)),
            pl.BlockSpec((TM1, H_), lambda i: (i, 0)),
            pl.BlockSpec((TM1, H_), lambda i: (i, 0)),
        ],
        out_shape=[
            jax.ShapeDtypeStruct((T_, E_), jnp.float32),
            jax.ShapeDtypeStruct((T_, H_), jnp.bfloat16),
            jax.ShapeDtypeStruct((T_, H_), jnp.bfloat16),
        ],
    )(hs2, gate_w, sg_w, su_w, sd_w)

    eid = jnp.argmax(logits, axis=1).astype(jnp.int32)
    oh = (eid[:, None] == jnp.arange(E_, dtype=jnp.int32)[None, :]).astype(jnp.int32)
    csum = jnp.cumsum(oh, axis=0)
    sizes = csum[-1]
    rank = jnp.take_along_axis(csum, eid[:, None], axis=1)[:, 0] - 1
    offsets = jnp.cumsum(sizes) - sizes
    pos_tok = offsets[eid] + rank

    ends = offsets + sizes
    t_all = jnp.arange(NT, dtype=jnp.int32)[:, None]
    e_all = jnp.arange(E_, dtype=jnp.int32)[None, :]
    lo2 = jnp.maximum(offsets[None, :], t_all * TM)
    hi2 = jnp.minimum(ends[None, :], (t_all + 1) * TM)
    fvalid = (lo2 < hi2).reshape(-1)
    srank = jnp.cumsum(fvalid.astype(jnp.int32)) - 1
    sidx = jnp.where(fvalid, srank, NSTEP)
    tt = jnp.broadcast_to(t_all, (NT, E_)).reshape(-1)
    ee = jnp.broadcast_to(e_all, (NT, E_)).reshape(-1)
    e_last = jnp.max(jnp.where(sizes > 0, jnp.arange(E_, dtype=jnp.int32), -1))
    init = jnp.stack(
        [jnp.full((NSTEP,), NT - 1, jnp.int32),
         jnp.zeros((NSTEP,), jnp.int32) + e_last,
         jnp.zeros((NSTEP,), jnp.int32),
         jnp.zeros((NSTEP,), jnp.int32)], axis=1)
    upd = jnp.stack(
        [tt, ee, (lo2 - t_all * TM).reshape(-1), (hi2 - t_all * TM).reshape(-1)],
        axis=1)
    steps = init.at[sidx].set(upd, mode="drop")
    step_t, step_e, step_lo, step_hi = (
        steps[:, 0], steps[:, 1], steps[:, 2], steps[:, 3])

    NC, NSC = 2, 16
    NW = NC * NSC
    mesh = plsc.VectorSubcoreMesh(
        core_axis_name="c", subcore_axis_name="s", num_cores=NC, num_subcores=NSC
    )

    CG = 32
    nch = T_ // NW // CG
    rows_w = T_ // NW

    def sc_scatter(tables, pos):
        ntab = len(tables)
        W = tables[0].shape[1]
        pos3 = pos.reshape(NW, nch, CG)

        @functools.partial(
            pl.kernel,
            out_type=tuple(
                jax.ShapeDtypeStruct((T_, W), jnp.float32) for _ in range(ntab)
            ),
            mesh=mesh,
            scratch_types=[
                pltpu.VMEM((nch, CG), jnp.int32),
                pltpu.VMEM((CG, W), jnp.float32),
                pltpu.SemaphoreType.DMA,
            ],
        )
        def _s(*refs):
            tabs = refs[:ntab]
            idx_hbm = refs[ntab]
            outs = refs[ntab + 1:2 * ntab + 1]
            idx_v, buf, sem = refs[2 * ntab + 1:]
            wid = lax.axis_index("s") * NC + lax.axis_index("c")
            base = wid * rows_w
            pltpu.sync_copy(idx_hbm.at[wid], idx_v)

            def body(ci, carry):
                for t in range(ntab):
                    pltpu.sync_copy(tabs[t].at[pl.ds(base + ci * CG, CG)], buf)
                    pltpu.async_copy(buf, outs[t].at[idx_v.at[ci]], sem).wait()
                return carry

            lax.fori_loop(0, nch, body, 0)

        return _s(*tables, pos3)

    def sc_gather(tables, idx):
        ntab = len(tables)
        W = tables[0].shape[1]

        @functools.partial(
            pl.kernel,
            out_type=tuple(
                jax.ShapeDtypeStruct((T_, W), jnp.float32) for _ in range(ntab)
            ),
            mesh=mesh,
            scratch_types=[
                pltpu.VMEM((rows_w,), jnp.int32),
                pltpu.VMEM((CG, W), jnp.float32),
                pltpu.SemaphoreType.DMA,
            ],
        )
        def _g(*refs):
            tabs = refs[:ntab]
            idx_hbm = refs[ntab]
            outs = refs[ntab + 1:2 * ntab + 1]
            idx_v, buf, sem = refs[2 * ntab + 1:]
            wid = lax.axis_index("s") * NC + lax.axis_index("c")
            base = wid * rows_w
            pltpu.sync_copy(idx_hbm.at[pl.ds(base, rows_w)], idx_v)

            def body(ci, carry):
                sl = pl.ds(ci * CG, CG)
                for t in range(ntab):
                    pltpu.async_copy(tabs[t].at[idx_v.at[sl]], buf, sem).wait()
                    pltpu.sync_copy(buf, outs[t].at[pl.ds(base + ci * CG, CG)])
                return carry

            lax.fori_loop(0, nch, body, 0)

        return _g(*tables, idx)

    def as_f32_view(a):
        return lax.bitcast_convert_type(a.reshape(T_, H_ // 2, 2), jnp.float32)

    def as_bf16_view(a):
        return lax.bitcast_convert_type(a, jnp.bfloat16).reshape(T_, H_)

    x_srt_v, sh_srt_v = sc_scatter(
        (as_f32_view(hsx), as_f32_view(shared)), pos_tok)
    x_srt = as_bf16_view(x_srt_v)
    sh_srt = as_bf16_view(sh_srt_v)

    final_srt = pl.pallas_call(
        _make_k3_body(TM),
        grid_spec=pltpu.PrefetchScalarGridSpec(
            num_scalar_prefetch=4,
            grid=(NSTEP,),
            in_specs=[
                pl.BlockSpec((TM, H_), lambda i, st, se, sl, sh: (st[i], 0)),
                pl.BlockSpec((TM, H_), lambda i, st, se, sl, sh: (st[i], 0)),
                pl.BlockSpec((1, Ir, H_), lambda i, st, se, sl, sh: (se[i], 0, 0)),
                pl.BlockSpec((1, Ir, H_), lambda i, st, se, sl, sh: (se[i], 0, 0)),
                pl.BlockSpec((1, H_, Ir), lambda i, st, se, sl, sh: (se[i], 0, 0)),
            ],
            out_specs=pl.BlockSpec((TM, H_), lambda i, st, se, sl, sh: (st[i], 0)),
        ),
        out_shape=jax.ShapeDtypeStruct((T_, H_), jnp.float32),
    )(step_t, step_e, step_lo, step_hi, x_srt, sh_srt, rg_w, ru_w, rd_w)

    (out2,) = sc_gather((final_srt,), pos_tok)

    return (out2.reshape(B_, S_, H_), logits.reshape(B_, S_, E_))


# speedup vs baseline: 2.0862x; 2.0862x over previous
import functools

import jax
import jax.numpy as jnp
from jax import lax
from jax.experimental import pallas as pl
from jax.experimental.pallas import tpu as pltpu
from jax.experimental.pallas import tpu_sc as plsc


def _sigmoid(x):
    return 1.0 / (1.0 + jnp.exp(-x))


def _k1_body(x_ref, gate_ref, sg_ref, su_ref, sd_ref, logits_ref, hsx_ref, shared_ref):
    x = x_ref[...]
    dn = (((1,), (1,)), ((), ()))
    logits = lax.dot_general(x, gate_ref[...], dn, preferred_element_type=jnp.float32)
    logits_ref[...] = logits
    score = _sigmoid(jnp.max(logits, axis=1, keepdims=True))
    hsx_ref[...] = x * score
    g = lax.dot_general(x, sg_ref[...], dn, preferred_element_type=jnp.float32)
    u = lax.dot_general(x, su_ref[...], dn, preferred_element_type=jnp.float32)
    h = g * _sigmoid(g) * u
    shared_ref[...] = lax.dot_general(h, sd_ref[...], dn, preferred_element_type=jnp.float32)


def _make_k3_body(TM):
    def _k3_body(step_t, step_e, step_lo, step_hi, x_ref, sh_ref, rg_ref, ru_ref,
                 rd_ref, out_ref):
        s = pl.program_id(0)
        lo = step_lo[s]
        hi = step_hi[s]
        dn = (((1,), (1,)), ((), ()))

        @pl.when(lo < hi)
        def _():
            x = x_ref[...]
            g = lax.dot_general(x, rg_ref[0], dn, preferred_element_type=jnp.float32)
            u = lax.dot_general(x, ru_ref[0], dn, preferred_element_type=jnp.float32)
            h = g * _sigmoid(g) * u
            y = (
                lax.dot_general(h, rd_ref[0], dn, preferred_element_type=jnp.float32)
                + sh_ref[...]
            )
            rows = lax.broadcasted_iota(jnp.int32, (x.shape[0], 1), 0)
            mask = (rows >= lo) & (rows < hi)
            out_ref[...] = jnp.where(mask, y, out_ref[...])

    return _k3_body


def kernel(hidden_states, gate_w, sg_w, su_w, sd_w, rg_w, ru_w, rd_w):
    B_, S_, H_ = hidden_states.shape
    E_ = gate_w.shape[0]
    Ish = sg_w.shape[0]
    Ir = rg_w.shape[1]
    T_ = B_ * S_

    TM1 = 256
    TM = 128
    NT = T_ // TM
    NSTEP = NT + E_ - 1

    hs2 = hidden_states.reshape(T_, H_)

    logits, hsx, shared = pl.pallas_call(
        _k1_body,
        grid=(T_ // TM1,),
        in_specs=[
            pl.BlockSpec((TM1, H_), lambda i: (i, 0)),
            pl.BlockSpec((E_, H_), lambda i: (0, 0)),
            pl.BlockSpec((Ish, H_), lambda i: (0, 0)),
            pl.BlockSpec((Ish, H_), lambda i: (0, 0)),
            pl.BlockSpec((H_, Ish), lambda i: (0, 0)),
        ],
        out_specs=[
            pl.BlockSpec((TM1, E_), lambda i: (i, 0)),
            pl.BlockSpec((TM1, H_), lambda i: (i, 0)),
            pl.BlockSpec((TM1, H_), lambda i: (i, 0)),
        ],
        out_shape=[
            jax.ShapeDtypeStruct((T_, E_), jnp.float32),
            jax.ShapeDtypeStruct((T_, H_), jnp.float32),
            jax.ShapeDtypeStruct((T_, H_), jnp.float32),
        ],
    )(hs2, gate_w, sg_w, su_w, sd_w)

    eid = jnp.argmax(logits, axis=1).astype(jnp.int32)
    oh = (eid[:, None] == jnp.arange(E_, dtype=jnp.int32)[None, :]).astype(jnp.int32)
    csum = jnp.cumsum(oh, axis=0)
    sizes = csum[-1]
    rank = jnp.take_along_axis(csum, eid[:, None], axis=1)[:, 0] - 1
    offsets = jnp.cumsum(sizes) - sizes
    pos_tok = offsets[eid] + rank

    ends = offsets + sizes
    t_all = jnp.arange(NT, dtype=jnp.int32)[:, None]
    e_all = jnp.arange(E_, dtype=jnp.int32)[None, :]
    lo2 = jnp.maximum(offsets[None, :], t_all * TM)
    hi2 = jnp.minimum(ends[None, :], (t_all + 1) * TM)
    fvalid = (lo2 < hi2).reshape(-1)
    srank = jnp.cumsum(fvalid.astype(jnp.int32)) - 1
    sidx = jnp.where(fvalid, srank, NSTEP)
    tt = jnp.broadcast_to(t_all, (NT, E_)).reshape(-1)
    ee = jnp.broadcast_to(e_all, (NT, E_)).reshape(-1)
    e_last = jnp.max(jnp.where(sizes > 0, jnp.arange(E_, dtype=jnp.int32), -1))
    init = jnp.stack(
        [jnp.full((NSTEP,), NT - 1, jnp.int32),
         jnp.zeros((NSTEP,), jnp.int32) + e_last,
         jnp.zeros((NSTEP,), jnp.int32),
         jnp.zeros((NSTEP,), jnp.int32)], axis=1)
    upd = jnp.stack(
        [tt, ee, (lo2 - t_all * TM).reshape(-1), (hi2 - t_all * TM).reshape(-1)],
        axis=1)
    steps = init.at[sidx].set(upd, mode="drop")
    step_t, step_e, step_lo, step_hi = (
        steps[:, 0], steps[:, 1], steps[:, 2], steps[:, 3])

    NC, NSC = 2, 16
    NW = NC * NSC
    mesh = plsc.VectorSubcoreMesh(
        core_axis_name="c", subcore_axis_name="s", num_cores=NC, num_subcores=NSC
    )

    CG = 32
    nch = T_ // NW // CG
    rows_w = T_ // NW

    def sc_scatter(tables, pos):
        ntab = len(tables)
        W = tables[0].shape[1]
        pos3 = pos.reshape(NW, nch, CG)

        @functools.partial(
            pl.kernel,
            out_type=tuple(
                jax.ShapeDtypeStruct((T_, W), jnp.float32) for _ in range(ntab)
            ),
            mesh=mesh,
            scratch_types=[
                pltpu.VMEM((nch, CG), jnp.int32),
                pltpu.VMEM((CG, W), jnp.float32),
                pltpu.SemaphoreType.DMA,
            ],
        )
        def _s(*refs):
            tabs = refs[:ntab]
            idx_hbm = refs[ntab]
            outs = refs[ntab + 1:2 * ntab + 1]
            idx_v, buf, sem = refs[2 * ntab + 1:]
            wid = lax.axis_index("s") * NC + lax.axis_index("c")
            base = wid * rows_w
            pltpu.sync_copy(idx_hbm.at[wid], idx_v)

            def body(ci, carry):
                for t in range(ntab):
                    pltpu.sync_copy(tabs[t].at[pl.ds(base + ci * CG, CG)], buf)
                    pltpu.async_copy(buf, outs[t].at[idx_v.at[ci]], sem).wait()
                return carry

            lax.fori_loop(0, nch, body, 0)

        return _s(*tables, pos3)

    def sc_gather(tables, idx):
        ntab = len(tables)
        W = tables[0].shape[1]

        @functools.partial(
            pl.kernel,
            out_type=tuple(
                jax.ShapeDtypeStruct((T_, W), jnp.float32) for _ in range(ntab)
            ),
            mesh=mesh,
            scratch_types=[
                pltpu.VMEM((rows_w,), jnp.int32),
                pltpu.VMEM((CG, W), jnp.float32),
                pltpu.SemaphoreType.DMA,
            ],
        )
        def _g(*refs):
            tabs = refs[:ntab]
            idx_hbm = refs[ntab]
            outs = refs[ntab + 1:2 * ntab + 1]
            idx_v, buf, sem = refs[2 * ntab + 1:]
            wid = lax.axis_index("s") * NC + lax.axis_index("c")
            base = wid * rows_w
            pltpu.sync_copy(idx_hbm.at[pl.ds(base, rows_w)], idx_v)

            def body(ci, carry):
                sl = pl.ds(ci * CG, CG)
                for t in range(ntab):
                    pltpu.async_copy(tabs[t].at[idx_v.at[sl]], buf, sem).wait()
                    pltpu.sync_copy(buf, outs[t].at[pl.ds(base + ci * CG, CG)])
                return carry

            lax.fori_loop(0, nch, body, 0)

        return _g(*tables, idx)

    x_srt, sh_srt = sc_scatter((hsx, shared), pos_tok)

    final_srt = pl.pallas_call(
        _make_k3_body(TM),
        grid_spec=pltpu.PrefetchScalarGridSpec(
            num_scalar_prefetch=4,
            grid=(NSTEP,),
            in_specs=[
                pl.BlockSpec((TM, H_), lambda i, st, se, sl, sh: (st[i], 0)),
                pl.BlockSpec((TM, H_), lambda i, st, se, sl, sh: (st[i], 0)),
                pl.BlockSpec((1, Ir, H_), lambda i, st, se, sl, sh: (se[i], 0, 0)),
                pl.BlockSpec((1, Ir, H_), lambda i, st, se, sl, sh: (se[i], 0, 0)),
                pl.BlockSpec((1, H_, Ir), lambda i, st, se, sl, sh: (se[i], 0, 0)),
            ],
            out_specs=pl.BlockSpec((TM, H_), lambda i, st, se, sl, sh: (st[i], 0)),
        ),
        out_shape=jax.ShapeDtypeStruct((T_, H_), jnp.float32),
    )(step_t, step_e, step_lo, step_hi, x_srt, sh_srt, rg_w, ru_w, rd_w)

    (out2,) = sc_gather((final_srt,), pos_tok)

    return (out2.reshape(B_, S_, H_), logits.reshape(B_, S_, E_))


# speedup vs baseline: 2.0875x; 1.0006x over previous
import functools

import jax
import jax.numpy as jnp
from jax import lax
from jax.experimental import pallas as pl
from jax.experimental.pallas import tpu as pltpu
from jax.experimental.pallas import tpu_sc as plsc


def _sigmoid(x):
    return 1.0 / (1.0 + jnp.exp(-x))


def _k1_body(x_ref, gate_ref, sg_ref, su_ref, sd_ref, logits_ref, hsx_ref, shared_ref):
    x = x_ref[...]
    dn = (((1,), (1,)), ((), ()))
    logits = lax.dot_general(x, gate_ref[...], dn, preferred_element_type=jnp.float32)
    logits_ref[...] = logits
    score = _sigmoid(jnp.max(logits, axis=1, keepdims=True))
    hsx_ref[...] = x * score
    g = lax.dot_general(x, sg_ref[...], dn, preferred_element_type=jnp.float32)
    u = lax.dot_general(x, su_ref[...], dn, preferred_element_type=jnp.float32)
    h = g * _sigmoid(g) * u
    shared_ref[...] = lax.dot_general(h, sd_ref[...], dn, preferred_element_type=jnp.float32)


def _make_k3_body(TM):
    def _k3_body(step_t, step_e, step_lo, step_hi, x_ref, sh_ref, rg_ref, ru_ref,
                 rd_ref, out_ref):
        s = pl.program_id(0)
        lo = step_lo[s]
        hi = step_hi[s]
        dn = (((1,), (1,)), ((), ()))

        @pl.when(lo < hi)
        def _():
            x = x_ref[...]
            g = lax.dot_general(x, rg_ref[0], dn, preferred_element_type=jnp.float32)
            u = lax.dot_general(x, ru_ref[0], dn, preferred_element_type=jnp.float32)
            h = g * _sigmoid(g) * u
            y = (
                lax.dot_general(h, rd_ref[0], dn, preferred_element_type=jnp.float32)
                + sh_ref[...]
            )
            full = (lo == 0) & (hi == TM)

            @pl.when(full)
            def _():
                out_ref[...] = y

            @pl.when(jnp.logical_not(full))
            def _():
                rows = lax.broadcasted_iota(jnp.int32, (TM, 1), 0)
                mask = (rows >= lo) & (rows < hi)
                out_ref[...] = jnp.where(mask, y, out_ref[...])

    return _k3_body


def kernel(hidden_states, gate_w, sg_w, su_w, sd_w, rg_w, ru_w, rd_w):
    B_, S_, H_ = hidden_states.shape
    E_ = gate_w.shape[0]
    Ish = sg_w.shape[0]
    Ir = rg_w.shape[1]
    T_ = B_ * S_

    TM1 = 256
    TM = 128
    NT = T_ // TM
    NSTEP = NT + E_ - 1

    hs2 = hidden_states.reshape(T_, H_)

    logits, hsx, shared = pl.pallas_call(
        _k1_body,
        grid=(T_ // TM1,),
        in_specs=[
            pl.BlockSpec((TM1, H_), lambda i: (i, 0)),
            pl.BlockSpec((E_, H_), lambda i: (0, 0)),
            pl.BlockSpec((Ish, H_), lambda i: (0, 0)),
            pl.BlockSpec((Ish, H_), lambda i: (0, 0)),
            pl.BlockSpec((H_, Ish), lambda i: (0, 0)),
        ],
        out_specs=[
            pl.BlockSpec((TM1, E_), lambda i: (i, 0)),
            pl.BlockSpec((TM1, H_), lambda i: (i, 0)),
            pl.BlockSpec((TM1, H_), lambda i: (i, 0)),
        ],
        out_shape=[
            jax.ShapeDtypeStruct((T_, E_), jnp.float32),
            jax.ShapeDtypeStruct((T_, H_), jnp.float32),
            jax.ShapeDtypeStruct((T_, H_), jnp.float32),
        ],
    )(hs2, gate_w, sg_w, su_w, sd_w)

    eid = jnp.argmax(logits, axis=1).astype(jnp.int32)
    oh = (eid[:, None] == jnp.arange(E_, dtype=jnp.int32)[None, :]).astype(jnp.int32)
    csum = jnp.cumsum(oh, axis=0)
    sizes = csum[-1]
    rank = jnp.take_along_axis(csum, eid[:, None], axis=1)[:, 0] - 1
    offsets = jnp.cumsum(sizes) - sizes
    pos_tok = offsets[eid] + rank

    ends = offsets + sizes
    t_all = jnp.arange(NT, dtype=jnp.int32)[:, None]
    e_all = jnp.arange(E_, dtype=jnp.int32)[None, :]
    lo2 = jnp.maximum(offsets[None, :], t_all * TM)
    hi2 = jnp.minimum(ends[None, :], (t_all + 1) * TM)
    fvalid = (lo2 < hi2).reshape(-1)
    srank = jnp.cumsum(fvalid.astype(jnp.int32)) - 1
    sidx = jnp.where(fvalid, srank, NSTEP)
    tt = jnp.broadcast_to(t_all, (NT, E_)).reshape(-1)
    ee = jnp.broadcast_to(e_all, (NT, E_)).reshape(-1)
    e_last = jnp.max(jnp.where(sizes > 0, jnp.arange(E_, dtype=jnp.int32), -1))
    init = jnp.stack(
        [jnp.full((NSTEP,), NT - 1, jnp.int32),
         jnp.zeros((NSTEP,), jnp.int32) + e_last,
         jnp.zeros((NSTEP,), jnp.int32),
         jnp.zeros((NSTEP,), jnp.int32)], axis=1)
    upd = jnp.stack(
        [tt, ee, (lo2 - t_all * TM).reshape(-1), (hi2 - t_all * TM).reshape(-1)],
        axis=1)
    steps = init.at[sidx].set(upd, mode="drop")
    step_t, step_e, step_lo, step_hi = (
        steps[:, 0], steps[:, 1], steps[:, 2], steps[:, 3])

    NC, NSC = 2, 16
    NW = NC * NSC
    mesh = plsc.VectorSubcoreMesh(
        core_axis_name="c", subcore_axis_name="s", num_cores=NC, num_subcores=NSC
    )

    CG = 32
    nch = T_ // NW // CG
    rows_w = T_ // NW

    def sc_scatter(tables, pos):
        ntab = len(tables)
        W = tables[0].shape[1]
        pos3 = pos.reshape(NW, nch, CG)

        @functools.partial(
            pl.kernel,
            out_type=tuple(
                jax.ShapeDtypeStruct((T_, W), jnp.float32) for _ in range(ntab)
            ),
            mesh=mesh,
            scratch_types=[
                pltpu.VMEM((nch, CG), jnp.int32),
                pltpu.VMEM((CG, W), jnp.float32),
                pltpu.SemaphoreType.DMA,
            ],
        )
        def _s(*refs):
            tabs = refs[:ntab]
            idx_hbm = refs[ntab]
            outs = refs[ntab + 1:2 * ntab + 1]
            idx_v, buf, sem = refs[2 * ntab + 1:]
            wid = lax.axis_index("s") * NC + lax.axis_index("c")
            base = wid * rows_w
            pltpu.sync_copy(idx_hbm.at[wid], idx_v)

            def body(ci, carry):
                for t in range(ntab):
                    pltpu.sync_copy(tabs[t].at[pl.ds(base + ci * CG, CG)], buf)
                    pltpu.async_copy(buf, outs[t].at[idx_v.at[ci]], sem).wait()
                return carry

            lax.fori_loop(0, nch, body, 0)

        return _s(*tables, pos3)

    def sc_gather(tables, idx):
        ntab = len(tables)
        W = tables[0].shape[1]

        @functools.partial(
            pl.kernel,
            out_type=tuple(
                jax.ShapeDtypeStruct((T_, W), jnp.float32) for _ in range(ntab)
            ),
            mesh=mesh,
            scratch_types=[
                pltpu.VMEM((rows_w,), jnp.int32),
                pltpu.VMEM((CG, W), jnp.float32),
                pltpu.SemaphoreType.DMA,
            ],
        )
        def _g(*refs):
            tabs = refs[:ntab]
            idx_hbm = refs[ntab]
            outs = refs[ntab + 1:2 * ntab + 1]
            idx_v, buf, sem = refs[2 * ntab + 1:]
            wid = lax.axis_index("s") * NC + lax.axis_index("c")
            base = wid * rows_w
            pltpu.sync_copy(idx_hbm.at[pl.ds(base, rows_w)], idx_v)

            def body(ci, carry):
                sl = pl.ds(ci * CG, CG)
                for t in range(ntab):
                    pltpu.async_copy(tabs[t].at[idx_v.at[sl]], buf, sem).wait()
                    pltpu.sync_copy(buf, outs[t].at[pl.ds(base + ci * CG, CG)])
                return carry

            lax.fori_loop(0, nch, body, 0)

        return _g(*tables, idx)

    x_srt, sh_srt = sc_scatter((hsx, shared), pos_tok)

    final_srt = pl.pallas_call(
        _make_k3_body(TM),
        grid_spec=pltpu.PrefetchScalarGridSpec(
            num_scalar_prefetch=4,
            grid=(NSTEP,),
            in_specs=[
                pl.BlockSpec((TM, H_), lambda i, st, se, sl, sh: (st[i], 0)),
                pl.BlockSpec((TM, H_), lambda i, st, se, sl, sh: (st[i], 0)),
                pl.BlockSpec((1, Ir, H_), lambda i, st, se, sl, sh: (se[i], 0, 0)),
                pl.BlockSpec((1, Ir, H_), lambda i, st, se, sl, sh: (se[i], 0, 0)),
                pl.BlockSpec((1, H_, Ir), lambda i, st, se, sl, sh: (se[i], 0, 0)),
            ],
            out_specs=pl.BlockSpec((TM, H_), lambda i, st, se, sl, sh: (st[i], 0)),
        ),
        out_shape=jax.ShapeDtypeStruct((T_, H_), jnp.float32),
    )(step_t, step_e, step_lo, step_hi, x_srt, sh_srt, rg_w, ru_w, rd_w)

    (out2,) = sc_gather((final_srt,), pos_tok)

    return (out2.reshape(B_, S_, H_), logits.reshape(B_, S_, E_))
